# Initial kernel scaffold; baseline (speedup 1.0000x reference)
import jax
import jax.numpy as jnp
from jax import lax
from jax.experimental import pallas as pl
from jax.experimental.pallas import tpu as pltpu
from jax.experimental.pallas import tpu_sc as plsc

N = 10000
D = 128
E = 320000
C_OUT = 10

NC = 2
NS = 16
NW = NC * NS
EPW = E // NW
K = 80
CHUNKS = EPW // K
RPT = N // NS
ZR = 125
NZ = RPT // ZR


def _sc_agg_body(h_hbm, src_hbm, dst_hbm, out_hbm,
                 idx_s, idx_d, rows, zbuf, acc, sem):
    c = lax.axis_index("c")
    s = lax.axis_index("s")
    wid = c * NS + s

    def _zrow(r, carry):
        for j in range(D // 16):
            zbuf[r, pl.ds(j * 16, 16)] = jnp.zeros((16,), jnp.float32)
        return carry

    lax.fori_loop(0, ZR, _zrow, 0)
    for t in range(NZ):
        pltpu.sync_copy(zbuf, acc.at[pl.ds(s * RPT + t * ZR, ZR)])
    plsc.subcore_barrier()

    def _chunk(g, carry):
        off = wid * EPW + g * K
        pltpu.sync_copy(src_hbm.at[pl.ds(off, K)], idx_s)
        pltpu.sync_copy(dst_hbm.at[pl.ds(off, K)], idx_d)
        pltpu.async_copy(h_hbm.at[idx_s], rows, sem).wait()
        pltpu.sync_copy(rows, acc.at[idx_d], add=True)
        return carry

    lax.fori_loop(0, CHUNKS, _chunk, 0)
    plsc.subcore_barrier()

    pltpu.sync_copy(acc.at[pl.ds(s * RPT, RPT)],
                    out_hbm.at[pl.ds(c * N + s * RPT, RPT)])


_sc_agg = pl.kernel(
    _sc_agg_body,
    out_type=jax.ShapeDtypeStruct((2 * N, D), jnp.float32),
    mesh=plsc.VectorSubcoreMesh(core_axis_name="c", subcore_axis_name="s"),
    scratch_types=[
        pltpu.VMEM((K,), jnp.int32),
        pltpu.VMEM((K,), jnp.int32),
        pltpu.VMEM((K, D), jnp.float32),
        pltpu.VMEM((ZR, D), jnp.float32),
        pltpu.VMEM_SHARED((N, D), jnp.float32),
        pltpu.SemaphoreType.DMA,
    ],
)

BLK = 1000
GRID = N // BLK

_row_spec = pl.BlockSpec((BLK, D), lambda i: (i, 0))
_pb_spec = pl.BlockSpec((BLK, D), lambda i: (i + GRID, 0))
_w_spec = pl.BlockSpec((D, D), lambda i: (0, 0))
_b_spec = pl.BlockSpec((1, D), lambda i: (0, 0))


def _mlp_body(h_ref, pa_ref, pb_ref, w1_ref, b1_ref, w2_ref, b2_ref, o_ref):
    m = h_ref[...] + pa_ref[...] + pb_ref[...]
    t = jnp.maximum(
        jnp.dot(m, w1_ref[...], preferred_element_type=jnp.float32)
        + b1_ref[...], 0.0)
    o_ref[...] = jnp.maximum(
        jnp.dot(t, w2_ref[...], preferred_element_type=jnp.float32)
        + b2_ref[...], 0.0)


_mlp1 = pl.pallas_call(
    _mlp_body,
    grid=(GRID,),
    in_specs=[_row_spec, _row_spec, _pb_spec, _w_spec, _b_spec, _w_spec,
              _b_spec],
    out_specs=_row_spec,
    out_shape=jax.ShapeDtypeStruct((N, D), jnp.float32),
)


def _mlp_pool_body(h_ref, pa_ref, pb_ref, w1_ref, b1_ref, w2_ref, b2_ref,
                   wo_ref, bo_ref, o_ref, acc_ref):
    i = pl.program_id(0)
    m = h_ref[...] + pa_ref[...] + pb_ref[...]
    t = jnp.maximum(
        jnp.dot(m, w1_ref[...], preferred_element_type=jnp.float32)
        + b1_ref[...], 0.0)
    h2 = jnp.maximum(
        jnp.dot(t, w2_ref[...], preferred_element_type=jnp.float32)
        + b2_ref[...], 0.0)
    ps = jnp.sum(h2, axis=0, keepdims=True)

    @pl.when(i == 0)
    def _():
        acc_ref[...] = ps

    @pl.when(i != 0)
    def _():
        acc_ref[...] = acc_ref[...] + ps

    @pl.when(i == GRID - 1)
    def _():
        o_ref[...] = (jnp.dot(acc_ref[...] * 2.0, wo_ref[...],
                              preferred_element_type=jnp.float32)
                      + bo_ref[...])


_mlp2 = pl.pallas_call(
    _mlp_pool_body,
    grid=(GRID,),
    in_specs=[_row_spec, _row_spec, _pb_spec, _w_spec, _b_spec, _w_spec,
              _b_spec,
              pl.BlockSpec((D, C_OUT), lambda i: (0, 0)),
              pl.BlockSpec((1, C_OUT), lambda i: (0, 0))],
    out_specs=pl.BlockSpec((1, C_OUT), lambda i: (0, 0)),
    out_shape=jax.ShapeDtypeStruct((1, C_OUT), jnp.float32),
    scratch_shapes=[pltpu.VMEM((1, D), jnp.float32)],
)


def kernel(x, edge_index, train_index, target_index, W1a, b1a, W2a, b2a,
           W1b, b1b, W2b, b2b, Wout, bout, att_train_k, att_target_k,
           att_train_q, att_target_q):
    ei = edge_index.astype(jnp.int32)
    src = ei[0]
    dst = ei[1]
    p1 = _sc_agg(x, src, dst)
    h1 = _mlp1(x, p1, p1, W1a, b1a.reshape(1, D), W2a, b2a.reshape(1, D))
    p2 = _sc_agg(h1, src, dst)
    out = _mlp2(h1, p2, p2, W1b, b1b.reshape(1, D), W2b, b2b.reshape(1, D),
                Wout, bout.reshape(1, C_OUT))
    return out


# speedup vs baseline: 4.7042x; 4.7042x over previous
import jax
import jax.numpy as jnp
from jax import lax
from jax.experimental import pallas as pl
from jax.experimental.pallas import tpu as pltpu
from jax.experimental.pallas import tpu_sc as plsc

N = 10000
D = 128
E = 320000
C_OUT = 10

NC = 2
NS = 16
NW = NC * NS
EPW = E // NW
K = 80
CHUNKS = EPW // K
RPT = 624
REM = N - NS * RPT
ZR = 208
NZ = RPT // ZR


def _sc_agg_body(h_hbm, src_hbm, dst_hbm, out_hbm,
                 idx_s, idx_d, rows, zbuf, acc, sem):
    c = lax.axis_index("c")
    s = lax.axis_index("s")
    wid = c * NS + s

    def _zrow(r, carry):
        for j in range(D // 16):
            zbuf[r, pl.ds(j * 16, 16)] = jnp.zeros((16,), jnp.float32)
        return carry

    lax.fori_loop(0, ZR, _zrow, 0)
    for t in range(NZ):
        pltpu.sync_copy(zbuf, acc.at[pl.ds(s * RPT + t * ZR, ZR)])

    @pl.when(s == 0)
    def _():
        pltpu.sync_copy(zbuf.at[pl.ds(0, REM)], acc.at[pl.ds(NS * RPT, REM)])

    plsc.subcore_barrier()

    def _chunk(g, carry):
        off = wid * EPW + g * K
        pltpu.sync_copy(src_hbm.at[pl.ds(off, K)], idx_s)
        pltpu.sync_copy(dst_hbm.at[pl.ds(off, K)], idx_d)
        pltpu.async_copy(h_hbm.at[idx_s], rows, sem).wait()
        pltpu.sync_copy(rows, acc.at[idx_d], add=True)
        return carry

    lax.fori_loop(0, CHUNKS, _chunk, 0)
    plsc.subcore_barrier()

    pltpu.sync_copy(acc.at[pl.ds(s * RPT, RPT)],
                    out_hbm.at[pl.ds(c * N + s * RPT, RPT)])

    @pl.when(s == 0)
    def _():
        pltpu.sync_copy(acc.at[pl.ds(NS * RPT, REM)],
                        out_hbm.at[pl.ds(c * N + NS * RPT, REM)])


_SC_AGG_CACHE = {}


def _sc_agg(h, src, dst):
    if "k" not in _SC_AGG_CACHE:
        _SC_AGG_CACHE["k"] = pl.kernel(
            _sc_agg_body,
            out_type=jax.ShapeDtypeStruct((2 * N, D), jnp.float32),
            mesh=plsc.VectorSubcoreMesh(core_axis_name="c",
                                        subcore_axis_name="s"),
            scratch_types=[
                pltpu.VMEM((K,), jnp.int32),
                pltpu.VMEM((K,), jnp.int32),
                pltpu.VMEM((K, D), jnp.float32),
                pltpu.VMEM((ZR, D), jnp.float32),
                pltpu.VMEM_SHARED((N, D), jnp.float32),
                pltpu.SemaphoreType.DMA,
            ],
        )
    return _SC_AGG_CACHE["k"](h, src, dst)

BLK = 1000
GRID = N // BLK

_row_spec = pl.BlockSpec((BLK, D), lambda i: (i, 0))
_pb_spec = pl.BlockSpec((BLK, D), lambda i: (i + GRID, 0))
_w_spec = pl.BlockSpec((D, D), lambda i: (0, 0))
_b_spec = pl.BlockSpec((1, D), lambda i: (0, 0))


def _mlp_body(h_ref, pa_ref, pb_ref, w1_ref, b1_ref, w2_ref, b2_ref, o_ref):
    m = h_ref[...] + pa_ref[...] + pb_ref[...]
    t = jnp.maximum(
        jnp.dot(m, w1_ref[...], preferred_element_type=jnp.float32)
        + b1_ref[...], 0.0)
    o_ref[...] = jnp.maximum(
        jnp.dot(t, w2_ref[...], preferred_element_type=jnp.float32)
        + b2_ref[...], 0.0)


_mlp1 = pl.pallas_call(
    _mlp_body,
    grid=(GRID,),
    in_specs=[_row_spec, _row_spec, _pb_spec, _w_spec, _b_spec, _w_spec,
              _b_spec],
    out_specs=_row_spec,
    out_shape=jax.ShapeDtypeStruct((N, D), jnp.float32),
)


def _mlp_pool_body(h_ref, pa_ref, pb_ref, w1_ref, b1_ref, w2_ref, b2_ref,
                   wo_ref, bo_ref, o_ref, acc_ref):
    i = pl.program_id(0)
    m = h_ref[...] + pa_ref[...] + pb_ref[...]
    t = jnp.maximum(
        jnp.dot(m, w1_ref[...], preferred_element_type=jnp.float32)
        + b1_ref[...], 0.0)
    h2 = jnp.maximum(
        jnp.dot(t, w2_ref[...], preferred_element_type=jnp.float32)
        + b2_ref[...], 0.0)
    ps = jnp.sum(h2, axis=0, keepdims=True)

    @pl.when(i == 0)
    def _():
        acc_ref[...] = ps

    @pl.when(i != 0)
    def _():
        acc_ref[...] = acc_ref[...] + ps

    @pl.when(i == GRID - 1)
    def _():
        o_ref[...] = (jnp.dot(acc_ref[...] * 2.0, wo_ref[...],
                              preferred_element_type=jnp.float32)
                      + bo_ref[...])


_mlp2 = pl.pallas_call(
    _mlp_pool_body,
    grid=(GRID,),
    in_specs=[_row_spec, _row_spec, _pb_spec, _w_spec, _b_spec, _w_spec,
              _b_spec,
              pl.BlockSpec((D, C_OUT), lambda i: (0, 0)),
              pl.BlockSpec((1, C_OUT), lambda i: (0, 0))],
    out_specs=pl.BlockSpec((1, C_OUT), lambda i: (0, 0)),
    out_shape=jax.ShapeDtypeStruct((1, C_OUT), jnp.float32),
    scratch_shapes=[pltpu.VMEM((1, D), jnp.float32)],
)


def kernel(x, edge_index, train_index, target_index, W1a, b1a, W2a, b2a,
           W1b, b1b, W2b, b2b, Wout, bout, att_train_k, att_target_k,
           att_train_q, att_target_q):
    ei = edge_index.astype(jnp.int32)
    src = ei[0]
    dst = ei[1]
    p1 = _sc_agg(x, src, dst)
    h1 = _mlp1(x, p1, p1, W1a, b1a.reshape(1, D), W2a, b2a.reshape(1, D))
    p2 = _sc_agg(h1, src, dst)
    out = _mlp2(h1, p2, p2, W1b, b1b.reshape(1, D), W2b, b2b.reshape(1, D),
                Wout, bout.reshape(1, C_OUT))
    return out


# speedup vs baseline: 10.5298x; 2.2384x over previous
import jax
import jax.numpy as jnp
from jax import lax
from jax.experimental import pallas as pl
from jax.experimental.pallas import tpu as pltpu
from jax.experimental.pallas import tpu_sc as plsc

N = 10000
D = 128
E = 320000
C_OUT = 10

NC = 2
NS = 16
NW = NC * NS
EPW = E // NW
K = 80
CHUNKS = EPW // K
RPT = 624
REM = N - NS * RPT


def _sc_agg_body(h_hbm, src_hbm, dst_hbm, out_hbm,
                 dst_all, sidx, rows0, rows1, acc,
                 isem0, isem1, isem2, isem3, gsem0, gsem1):
    c = lax.axis_index("c")
    s = lax.axis_index("s")
    wid = c * NS + s

    cp_d = pltpu.async_copy(dst_hbm.at[wid], dst_all, gsem0)
    cp_h = pltpu.async_copy(h_hbm.at[pl.ds(s * RPT, RPT)],
                            acc.at[pl.ds(s * RPT, RPT)], gsem1)

    @pl.when(s == 0)
    def _():
        pltpu.async_copy(h_hbm.at[pl.ds(NS * RPT, REM)],
                         acc.at[pl.ds(NS * RPT, REM)], gsem1).wait()

    cp_d.wait()
    cp_h.wait()
    plsc.subcore_barrier()

    isems = (isem0, isem1, isem2, isem3)

    def _iload(g, slot):
        pltpu.async_copy(src_hbm.at[wid, g], sidx.at[slot], isems[slot])

    def _iwait(slot):
        pltpu.make_async_copy(src_hbm.at[0, 0], sidx.at[slot],
                              isems[slot]).wait()

    def _gather(slot, buf, sem):
        pltpu.async_copy(h_hbm.at[sidx.at[slot]], buf, sem)

    def _gwait(buf, sem):
        pltpu.make_async_copy(h_hbm.at[pl.ds(0, K)], buf, sem).wait()

    def _scat(g, buf):
        pltpu.sync_copy(buf, acc.at[dst_all.at[g]], add=True)

    for slot in range(4):
        _iload(slot, slot)
    _iwait(0)
    _gather(0, rows0, gsem0)
    _iwait(1)
    _gather(1, rows1, gsem1)

    def _quad(q, carry):
        g = 4 * q
        for j in range(4):
            buf, gsem = (rows0, gsem0) if j % 2 == 0 else (rows1, gsem1)
            _gwait(buf, gsem)
            _scat(g + j, buf)

            @pl.when(g + j + 4 < CHUNKS)
            def _():
                _iload(g + j + 4, j)

            @pl.when(g + j + 2 < CHUNKS)
            def _():
                _iwait((j + 2) % 4)
                _gather((j + 2) % 4, buf, gsem)

        return carry

    lax.fori_loop(0, CHUNKS // 4, _quad, 0)
    for g in range((CHUNKS // 4) * 4, CHUNKS):
        buf, gsem = (rows0, gsem0) if g % 2 == 0 else (rows1, gsem1)
        _gwait(buf, gsem)
        _scat(g, buf)
    plsc.subcore_barrier()

    pltpu.sync_copy(acc.at[pl.ds(s * RPT, RPT)],
                    out_hbm.at[pl.ds(c * N + s * RPT, RPT)])

    @pl.when(s == 0)
    def _():
        pltpu.sync_copy(acc.at[pl.ds(NS * RPT, REM)],
                        out_hbm.at[pl.ds(c * N + NS * RPT, REM)])


_SC_AGG_CACHE = {}


def _sc_agg(h, src, dst):
    if "k" not in _SC_AGG_CACHE:
        _SC_AGG_CACHE["k"] = pl.kernel(
            _sc_agg_body,
            out_type=jax.ShapeDtypeStruct((2 * N, D), jnp.float32),
            mesh=plsc.VectorSubcoreMesh(core_axis_name="c",
                                        subcore_axis_name="s"),
            scratch_types=[
                pltpu.VMEM((CHUNKS, K), jnp.int32),
                pltpu.VMEM((4, K), jnp.int32),
                pltpu.VMEM((K, D), jnp.float32),
                pltpu.VMEM((K, D), jnp.float32),
                pltpu.VMEM_SHARED((N, D), jnp.float32),
                pltpu.SemaphoreType.DMA,
                pltpu.SemaphoreType.DMA,
                pltpu.SemaphoreType.DMA,
                pltpu.SemaphoreType.DMA,
                pltpu.SemaphoreType.DMA,
                pltpu.SemaphoreType.DMA,
            ],
        )
    return _SC_AGG_CACHE["k"](h, src, dst)

BLK = 1000
GRID = N // BLK

_row_spec = pl.BlockSpec((BLK, D), lambda i: (i, 0))
_pb_spec = pl.BlockSpec((BLK, D), lambda i: (i + GRID, 0))
_w_spec = pl.BlockSpec((D, D), lambda i: (0, 0))
_b_spec = pl.BlockSpec((1, D), lambda i: (0, 0))


def _mlp_body(h_ref, pa_ref, pb_ref, w1_ref, b1_ref, w2_ref, b2_ref, o_ref):
    m = pa_ref[...] + pb_ref[...] - h_ref[...]
    t = jnp.maximum(
        jnp.dot(m, w1_ref[...], preferred_element_type=jnp.float32)
        + b1_ref[...], 0.0)
    o_ref[...] = jnp.maximum(
        jnp.dot(t, w2_ref[...], preferred_element_type=jnp.float32)
        + b2_ref[...], 0.0)


_mlp1 = pl.pallas_call(
    _mlp_body,
    grid=(GRID,),
    in_specs=[_row_spec, _row_spec, _pb_spec, _w_spec, _b_spec, _w_spec,
              _b_spec],
    out_specs=_row_spec,
    out_shape=jax.ShapeDtypeStruct((N, D), jnp.float32),
)


def _mlp_pool_body(h_ref, pa_ref, pb_ref, w1_ref, b1_ref, w2_ref, b2_ref,
                   wo_ref, bo_ref, o_ref, acc_ref):
    i = pl.program_id(0)
    m = pa_ref[...] + pb_ref[...] - h_ref[...]
    t = jnp.maximum(
        jnp.dot(m, w1_ref[...], preferred_element_type=jnp.float32)
        + b1_ref[...], 0.0)
    h2 = jnp.maximum(
        jnp.dot(t, w2_ref[...], preferred_element_type=jnp.float32)
        + b2_ref[...], 0.0)
    ps = jnp.sum(h2, axis=0, keepdims=True)

    @pl.when(i == 0)
    def _():
        acc_ref[...] = ps

    @pl.when(i != 0)
    def _():
        acc_ref[...] = acc_ref[...] + ps

    @pl.when(i == GRID - 1)
    def _():
        o_ref[...] = (jnp.dot(acc_ref[...] * 2.0, wo_ref[...],
                              preferred_element_type=jnp.float32)
                      + bo_ref[...])


_mlp2 = pl.pallas_call(
    _mlp_pool_body,
    grid=(GRID,),
    in_specs=[_row_spec, _row_spec, _pb_spec, _w_spec, _b_spec, _w_spec,
              _b_spec,
              pl.BlockSpec((D, C_OUT), lambda i: (0, 0)),
              pl.BlockSpec((1, C_OUT), lambda i: (0, 0))],
    out_specs=pl.BlockSpec((1, C_OUT), lambda i: (0, 0)),
    out_shape=jax.ShapeDtypeStruct((1, C_OUT), jnp.float32),
    scratch_shapes=[pltpu.VMEM((1, D), jnp.float32)],
)


def kernel(x, edge_index, train_index, target_index, W1a, b1a, W2a, b2a,
           W1b, b1b, W2b, b2b, Wout, bout, att_train_k, att_target_k,
           att_train_q, att_target_q):
    ei = edge_index.astype(jnp.int32).reshape(2, NW, CHUNKS, K)
    src = ei[0]
    dst = ei[1]
    p1 = _sc_agg(x, src, dst)
    h1 = _mlp1(x, p1, p1, W1a, b1a.reshape(1, D), W2a, b2a.reshape(1, D))
    p2 = _sc_agg(h1, src, dst)
    out = _mlp2(h1, p2, p2, W1b, b1b.reshape(1, D), W2b, b2b.reshape(1, D),
                Wout, bout.reshape(1, C_OUT))
    return out
